# baseline (speedup 1.0000x reference)
import functools
import math

import jax
import jax.numpy as jnp
from jax import lax
from jax.experimental import pallas as pl
from jax.experimental.pallas import tpu as pltpu
from jax.experimental.pallas import tpu_sc as plsc

_N = 10000
_E = 320000
_NFEAT = 128
_NHID = 128
_NCLASS = 64
_NLAYERS = 4
_LAMDA = 0.5
_ALPHA = 0.1

_NC = 2
_NS = 16
_NW = _NC * _NS
_NPAD = 10240
_EPT = _NPAD
_NBKT = 5
_BROWS = 2048
_CHUNK = 48
_BCAP = 48
_SEG = _BCAP * _CHUNK
_TSEG = _NBKT * _SEG
_NBUF = 2
_LANES = 16
_RPT = _NPAD // _NS


def _bcast_lane(v16, k):
    return lax.gather(
        v16, jnp.full((_LANES, 1), k, jnp.int32),
        lax.GatherDimensionNumbers(
            offset_dims=(), collapsed_slice_dims=(0,), start_index_map=(0,)),
        slice_sizes=(1,),
        mode=lax.GatherScatterMode.PROMISE_IN_BOUNDS)


_bcast_lane_i = _bcast_lane


def _gather16(v, idx_const):
    return lax.gather(
        v, idx_const[:, None],
        lax.GatherDimensionNumbers(
            offset_dims=(), collapsed_slice_dims=(0,), start_index_map=(0,)),
        slice_sizes=(1,),
        mode=lax.GatherScatterMode.PROMISE_IN_BOUNDS)


def _prefix16(mi):
    lane = jnp.arange(_LANES, dtype=jnp.int32)
    pc = mi
    for k in (1, 2, 4, 8):
        sh = _gather16(pc, jnp.maximum(lane - k, 0))
        pc = pc + jnp.where(lane >= k, sh, 0)
    return pc


def _presort_body(sp, dp, wpf, os_h, od_h, ow_h,
                  ebs, ebd, ebw, dstg, vs, vd, vw, zbi, zbf,
                  ts_sh, td_sh, tw_sh, lsem, psem):
    c = lax.axis_index("c")
    s = lax.axis_index("s")
    wid = c * _NS + s
    tbase = s * _TSEG

    def _z(i, cr):
        zbi[pl.ds(i * _LANES, _LANES)] = jnp.zeros((_LANES,), jnp.int32)
        zbf[pl.ds(i * _LANES, _LANES)] = jnp.zeros((_LANES,), jnp.float32)
        return cr
    lax.fori_loop(0, _SEG // _LANES, _z, 0)
    for q in range(_NBKT):
        pltpu.async_copy(zbi, ts_sh.at[pl.ds(tbase + q * _SEG, _SEG)], lsem)
        pltpu.async_copy(zbi, td_sh.at[pl.ds(tbase + q * _SEG, _SEG)], lsem)
        pltpu.async_copy(zbf, tw_sh.at[pl.ds(tbase + q * _SEG, _SEG)], lsem)
    for q in range(_NBKT):
        for _ in range(3):
            pltpu.make_async_copy(
                zbi, ts_sh.at[pl.ds(tbase, _SEG)], lsem).wait()

    bases = (jnp.zeros((_LANES,), jnp.int32),) * _NBKT
    half = _EPT // 2
    for hh in range(2):
        pltpu.sync_copy(sp.at[wid, pl.ds(hh * half, half)], ebs)
        pltpu.sync_copy(dp.at[wid, pl.ds(hh * half, half)], ebd)
        pltpu.sync_copy(wpf.at[wid, pl.ds(hh * half, half)], ebw)

        def _grp(go, carry, first=(hh == 0)):
            for k in range(2):
                off = (go * 2 + k) * _LANES
                sv = ebs[pl.ds(off, _LANES)]
                dv = ebd[pl.ds(off, _LANES)]
                wv = ebw[pl.ds(off, _LANES)]
                key = lax.shift_right_logical(sv, 11)
                sloc = lax.bitwise_and(sv, jnp.int32(_BROWS - 1))
                dest = jnp.zeros((_LANES,), jnp.int32)
                nb = []
                for q in range(_NBKT):
                    m = key == q
                    mi = jnp.where(m, jnp.int32(1), jnp.int32(0))
                    pc = _prefix16(mi)
                    dest = jnp.where(
                        m, (s * _NBKT + q) * _SEG + carry[q] + pc - 1, dest)
                    nb.append(carry[q] + _bcast_lane_i(pc, _LANES - 1))
                carry = tuple(nb)

                def _drain():
                    pltpu.make_async_copy(
                        vs.at[k], ts_sh.at[dstg.at[k]], psem.at[k]).wait()
                    pltpu.make_async_copy(
                        vd.at[k], td_sh.at[dstg.at[k]], psem.at[k]).wait()
                    pltpu.make_async_copy(
                        vw.at[k], tw_sh.at[dstg.at[k]], psem.at[k]).wait()
                if first:
                    @pl.when(go > 0)
                    def _():
                        _drain()
                else:
                    _drain()
                dstg[k, pl.ds(0, _LANES)] = dest
                vs[k, pl.ds(0, _LANES)] = sloc
                vd[k, pl.ds(0, _LANES)] = dv
                vw[k, pl.ds(0, _LANES)] = wv
                pltpu.async_copy(vs.at[k], ts_sh.at[dstg.at[k]], psem.at[k])
                pltpu.async_copy(vd.at[k], td_sh.at[dstg.at[k]], psem.at[k])
                pltpu.async_copy(vw.at[k], tw_sh.at[dstg.at[k]], psem.at[k])
            return carry
        bases = lax.fori_loop(0, half // (2 * _LANES), _grp, bases)

    for k in range(2):
        pltpu.make_async_copy(vs.at[k], ts_sh.at[dstg.at[k]], psem.at[k]).wait()
        pltpu.make_async_copy(vd.at[k], td_sh.at[dstg.at[k]], psem.at[k]).wait()
        pltpu.make_async_copy(vw.at[k], tw_sh.at[dstg.at[k]], psem.at[k]).wait()

    pltpu.sync_copy(ts_sh.at[pl.ds(tbase, _TSEG)],
                    os_h.at[pl.ds(wid * _TSEG, _TSEG)])
    pltpu.sync_copy(td_sh.at[pl.ds(tbase, _TSEG)],
                    od_h.at[pl.ds(wid * _TSEG, _TSEG)])
    pltpu.sync_copy(tw_sh.at[pl.ds(tbase, _TSEG)],
                    ow_h.at[pl.ds(wid * _TSEG, _TSEG)])


def _make_presort():
    mesh = plsc.VectorSubcoreMesh(core_axis_name="c", subcore_axis_name="s")
    n = _NW * _TSEG
    return pl.kernel(
        _presort_body,
        out_type=(jax.ShapeDtypeStruct((n,), jnp.int32),
                  jax.ShapeDtypeStruct((n,), jnp.int32),
                  jax.ShapeDtypeStruct((n,), jnp.float32)),
        mesh=mesh,
        scratch_types=[
            pltpu.VMEM((_EPT // 2,), jnp.int32),
            pltpu.VMEM((_EPT // 2,), jnp.int32),
            pltpu.VMEM((_EPT // 2,), jnp.float32),
            pltpu.VMEM((2, _LANES), jnp.int32),
            pltpu.VMEM((2, _LANES), jnp.int32),
            pltpu.VMEM((2, _LANES), jnp.int32),
            pltpu.VMEM((2, _LANES), jnp.float32),
            pltpu.VMEM((_SEG,), jnp.int32),
            pltpu.VMEM((_SEG,), jnp.float32),
            pltpu.VMEM_SHARED((_NS * _TSEG,), jnp.int32),
            pltpu.VMEM_SHARED((_NS * _TSEG,), jnp.int32),
            pltpu.VMEM_SHARED((_NS * _TSEG,), jnp.float32),
            pltpu.SemaphoreType.DMA,
            pltpu.SemaphoreType.DMA((2,)),
        ],
    )


def _spmm_body(h_hbm, st, wt, dt, out_hbm,
               sring, wring, dring, rows, acc_sh, hstage,
               tsem, dsem, gsem, ssem, hsem):
    c = lax.axis_index("c")
    s = lax.axis_index("s")
    wid = c * _NS + s

    def _zrow(i, carry):
        for j in range(_NHID // _LANES):
            rows[0, i, pl.ds(j * _LANES, _LANES)] = jnp.zeros((_LANES,),
                                                              jnp.float32)
        return carry
    lax.fori_loop(0, _CHUNK, _zrow, 0)
    for q in range(_RPT // 40):
        pltpu.async_copy(rows.at[0, pl.ds(0, 40)],
                         acc_sh.at[pl.ds(s * _RPT + q * 40, 40)],
                         ssem.at[0])
    for q in range(_RPT // 40):
        pltpu.make_async_copy(rows.at[0, pl.ds(0, 40)],
                              acc_sh.at[pl.ds(s * _RPT, 40)],
                              ssem.at[0]).wait()
    plsc.subcore_barrier()

    def _scale_chunk(b):
        for g in range(_CHUNK // _LANES):
            w16 = wring[b, pl.ds(g * _LANES, _LANES)]
            for k in range(_LANES):
                wk = _bcast_lane(w16, k)
                e = g * _LANES + k
                for j in range(_NHID // _LANES):
                    sl = pl.ds(j * _LANES, _LANES)
                    rows[b, e, sl] = rows[b, e, sl] * wk

    for q in range(_NBKT):
        pltpu.async_copy(h_hbm.at[pl.ds(q * _BROWS + s * 128, 128)],
                         hstage.at[pl.ds(s * 128, 128)], hsem)
        pltpu.make_async_copy(h_hbm.at[pl.ds(q * _BROWS + s * 128, 128)],
                              hstage.at[pl.ds(s * 128, 128)], hsem).wait()
        plsc.subcore_barrier()

        for b in range(_NBUF):
            pltpu.async_copy(st.at[wid, q, b], sring.at[b], tsem.at[b])
            pltpu.async_copy(wt.at[wid, q, b], wring.at[b], tsem.at[b])
        for b in range(_NBUF - 1):
            pltpu.async_copy(dt.at[wid, q, b], dring.at[b], dsem.at[b])
        for b in range(_NBUF - 1):
            pltpu.make_async_copy(st.at[wid, q, b], sring.at[b],
                                  tsem.at[b]).wait()
            pltpu.make_async_copy(wt.at[wid, q, b], wring.at[b],
                                  tsem.at[b]).wait()
            pltpu.async_copy(hstage.at[sring.at[b]], rows.at[b], gsem.at[b])

        def _outer(to, carry):
            for b in range(_NBUF):
                t = to * _NBUF + b
                pb = (b - 1) % _NBUF
                pltpu.make_async_copy(hstage.at[sring.at[b]], rows.at[b],
                                      gsem.at[b]).wait()
                _scale_chunk(b)
                @pl.when(t + _NBUF < _BCAP)
                def _():
                    pltpu.async_copy(st.at[wid, q, t + _NBUF], sring.at[b],
                                     tsem.at[b])
                    pltpu.async_copy(wt.at[wid, q, t + _NBUF], wring.at[b],
                                     tsem.at[b])
                def _drain_prev():
                    pltpu.make_async_copy(rows.at[pb],
                                          acc_sh.at[dring.at[pb]],
                                          ssem.at[pb]).wait()
                if b == 0:
                    @pl.when(to > 0)
                    def _():
                        _drain_prev()
                else:
                    _drain_prev()
                @pl.when(t + _NBUF - 1 < _BCAP)
                def _():
                    pltpu.async_copy(dt.at[wid, q, t + _NBUF - 1],
                                     dring.at[pb], dsem.at[pb])
                pltpu.make_async_copy(dt.at[wid, q, t], dring.at[b],
                                      dsem.at[b]).wait()
                pltpu.async_copy(rows.at[b], acc_sh.at[dring.at[b]],
                                 ssem.at[b], add=True)
                @pl.when(t + _NBUF - 1 < _BCAP)
                def _():
                    pltpu.make_async_copy(st.at[wid, q, t + _NBUF - 1],
                                          sring.at[pb], tsem.at[pb]).wait()
                    pltpu.make_async_copy(wt.at[wid, q, t + _NBUF - 1],
                                          wring.at[pb], tsem.at[pb]).wait()
                    pltpu.async_copy(hstage.at[sring.at[pb]], rows.at[pb],
                                     gsem.at[pb])
            return carry
        lax.fori_loop(0, _BCAP // _NBUF, _outer, 0)

        pltpu.make_async_copy(rows.at[(_BCAP - 1) % _NBUF],
                              acc_sh.at[dring.at[(_BCAP - 1) % _NBUF]],
                              ssem.at[(_BCAP - 1) % _NBUF]).wait()
        plsc.subcore_barrier()

    pltpu.sync_copy(acc_sh.at[pl.ds(s * _RPT, _RPT)],
                    out_hbm.at[c, pl.ds(s * _RPT, _RPT)])


def _make_spmm():
    mesh = plsc.VectorSubcoreMesh(core_axis_name="c", subcore_axis_name="s")
    return pl.kernel(
        _spmm_body,
        out_type=jax.ShapeDtypeStruct((_NC, _NPAD, _NHID), jnp.float32),
        mesh=mesh,
        scratch_types=[
            pltpu.VMEM((_NBUF, _CHUNK), jnp.int32),
            pltpu.VMEM((_NBUF, _CHUNK), jnp.float32),
            pltpu.VMEM((_NBUF, _CHUNK), jnp.int32),
            pltpu.VMEM((_NBUF, _CHUNK, _NHID), jnp.float32),
            pltpu.VMEM_SHARED((_NPAD, _NHID), jnp.float32),
            pltpu.VMEM_SHARED((_BROWS, _NHID), jnp.float32),
            pltpu.SemaphoreType.DMA((_NBUF,)),
            pltpu.SemaphoreType.DMA((_NBUF,)),
            pltpu.SemaphoreType.DMA((_NBUF,)),
            pltpu.SemaphoreType.DMA((_NBUF,)),
            pltpu.SemaphoreType.DMA,
        ],
    )


_BN = 1024
_BNF = 1000


def _fc0_body(x_ref, w_ref, b_ref, o_ref):
    t = jnp.dot(x_ref[...], w_ref[...], preferred_element_type=jnp.float32)
    o_ref[...] = jnp.maximum(t + b_ref[...], 0.0)


def _dense_body(theta, p_ref, h0_ref, h_ref, w_ref, o_ref):
    sup = (1.0 - _ALPHA) * (p_ref[0] + p_ref[1]) + _ALPHA * h0_ref[...]
    t = jnp.dot(sup, w_ref[...], preferred_element_type=jnp.float32)
    o_ref[...] = jnp.maximum(theta * t + (1.0 - theta) * sup + h_ref[...], 0.0)


def _final_body(h_ref, w_ref, b_ref, o_ref):
    t = jnp.dot(h_ref[...], w_ref[...], preferred_element_type=jnp.float32)
    o_ref[...] = jax.nn.sigmoid(t + b_ref[...])


def _fc0(x, W0, b0):
    return pl.pallas_call(
        _fc0_body,
        grid=(_NPAD // _BN,),
        in_specs=[
            pl.BlockSpec((_BN, _NFEAT), lambda i: (i, 0)),
            pl.BlockSpec((_NFEAT, _NHID), lambda i: (0, 0)),
            pl.BlockSpec((1, _NHID), lambda i: (0, 0)),
        ],
        out_specs=pl.BlockSpec((_BN, _NHID), lambda i: (i, 0)),
        out_shape=jax.ShapeDtypeStruct((_NPAD, _NHID), jnp.float32),
    )(x, W0, b0.reshape(1, _NHID))


def _dense(p, h0, h, W, theta):
    return pl.pallas_call(
        functools.partial(_dense_body, theta),
        grid=(_NPAD // _BN,),
        in_specs=[
            pl.BlockSpec((_NC, _BN, _NHID), lambda i: (0, i, 0)),
            pl.BlockSpec((_BN, _NHID), lambda i: (i, 0)),
            pl.BlockSpec((_BN, _NHID), lambda i: (i, 0)),
            pl.BlockSpec((_NHID, _NHID), lambda i: (0, 0)),
        ],
        out_specs=pl.BlockSpec((_BN, _NHID), lambda i: (i, 0)),
        out_shape=jax.ShapeDtypeStruct((_NPAD, _NHID), jnp.float32),
    )(p, h0, h, W)


def _final(h, Wout, bout):
    return pl.pallas_call(
        _final_body,
        grid=(_N // _BNF,),
        in_specs=[
            pl.BlockSpec((_BNF, _NHID), lambda i: (i, 0)),
            pl.BlockSpec((_NHID, _NCLASS), lambda i: (0, 0)),
            pl.BlockSpec((1, _NCLASS), lambda i: (0, 0)),
        ],
        out_specs=pl.BlockSpec((_BNF, _NCLASS), lambda i: (i, 0)),
        out_shape=jax.ShapeDtypeStruct((_N, _NCLASS), jnp.float32),
    )(h, Wout, bout.reshape(1, _NCLASS))


def kernel(x, edge_index, edge_weight, W0, b0, Wc, Wout, bout):
    ppt = _EPT - _E // _NW
    sp = jnp.concatenate(
        [edge_index[0].reshape(_NW, _E // _NW),
         jnp.full((_NW, ppt), _NPAD - 1, jnp.int32)], axis=1)
    dp = jnp.concatenate(
        [edge_index[1].reshape(_NW, _E // _NW),
         jnp.zeros((_NW, ppt), jnp.int32)], axis=1)
    wpf = jnp.concatenate(
        [edge_weight.reshape(_NW, _E // _NW),
         jnp.zeros((_NW, ppt), jnp.float32)], axis=1)
    st_f, dt_f, wt_f = _make_presort()(sp, dp, wpf)
    st = st_f.reshape(_NW, _NBKT, _BCAP, _CHUNK)
    dt = dt_f.reshape(_NW, _NBKT, _BCAP, _CHUNK)
    wt = wt_f.reshape(_NW, _NBKT, _BCAP, _CHUNK)

    xpad = jnp.concatenate(
        [x, jnp.zeros((_NPAD - _N, _NFEAT), jnp.float32)], axis=0)
    spmm = _make_spmm()
    h = _fc0(xpad, W0, b0)
    h0 = h
    for i in range(_NLAYERS):
        theta = math.log(_LAMDA / (i + 1) + 1.0)
        p = spmm(h, st, wt, dt)
        h = _dense(p, h0, h, Wc[i], theta)
    return _final(h, Wout, bout)


# speedup vs baseline: 1.0342x; 1.0342x over previous
import functools
import math

import jax
import jax.numpy as jnp
from jax import lax
from jax.experimental import pallas as pl
from jax.experimental.pallas import tpu as pltpu
from jax.experimental.pallas import tpu_sc as plsc

_N = 10000
_E = 320000
_NFEAT = 128
_NHID = 128
_NCLASS = 64
_NLAYERS = 4
_LAMDA = 0.5
_ALPHA = 0.1

_NC = 2
_NS = 16
_NW = _NC * _NS
_NPAD = 10240
_EPT = _NPAD
_NBKT = 5
_BROWS = 2048
_CHUNK = 32
_BCAP = 72
_SEG = _BCAP * _CHUNK
_TSEG = _NBKT * _SEG
_NBUF = 3
_LANES = 16
_RPT = _NPAD // _NS


def _bcast_lane(v16, k):
    return lax.gather(
        v16, jnp.full((_LANES, 1), k, jnp.int32),
        lax.GatherDimensionNumbers(
            offset_dims=(), collapsed_slice_dims=(0,), start_index_map=(0,)),
        slice_sizes=(1,),
        mode=lax.GatherScatterMode.PROMISE_IN_BOUNDS)


_bcast_lane_i = _bcast_lane


def _gather16(v, idx_const):
    return lax.gather(
        v, idx_const[:, None],
        lax.GatherDimensionNumbers(
            offset_dims=(), collapsed_slice_dims=(0,), start_index_map=(0,)),
        slice_sizes=(1,),
        mode=lax.GatherScatterMode.PROMISE_IN_BOUNDS)


def _prefix16(mi):
    lane = jnp.arange(_LANES, dtype=jnp.int32)
    pc = mi
    for k in (1, 2, 4, 8):
        sh = _gather16(pc, jnp.maximum(lane - k, 0))
        pc = pc + jnp.where(lane >= k, sh, 0)
    return pc


def _presort_body(sp, dp, wpf, os_h, od_h, ow_h,
                  ebs, ebd, ebw, dstg, vs, vd, vw, zbi, zbf,
                  ts_sh, td_sh, tw_sh, lsem, psem):
    c = lax.axis_index("c")
    s = lax.axis_index("s")
    wid = c * _NS + s
    tbase = s * _TSEG

    def _z(i, cr):
        zbi[pl.ds(i * _LANES, _LANES)] = jnp.zeros((_LANES,), jnp.int32)
        zbf[pl.ds(i * _LANES, _LANES)] = jnp.zeros((_LANES,), jnp.float32)
        return cr
    lax.fori_loop(0, _SEG // _LANES, _z, 0)
    for q in range(_NBKT):
        pltpu.async_copy(zbi, ts_sh.at[pl.ds(tbase + q * _SEG, _SEG)], lsem)
        pltpu.async_copy(zbi, td_sh.at[pl.ds(tbase + q * _SEG, _SEG)], lsem)
        pltpu.async_copy(zbf, tw_sh.at[pl.ds(tbase + q * _SEG, _SEG)], lsem)
    for q in range(_NBKT):
        for _ in range(3):
            pltpu.make_async_copy(
                zbi, ts_sh.at[pl.ds(tbase, _SEG)], lsem).wait()

    bases = (jnp.zeros((_LANES,), jnp.int32),) * _NBKT
    half = _EPT // 2
    for hh in range(2):
        pltpu.sync_copy(sp.at[wid, pl.ds(hh * half, half)], ebs)
        pltpu.sync_copy(dp.at[wid, pl.ds(hh * half, half)], ebd)
        pltpu.sync_copy(wpf.at[wid, pl.ds(hh * half, half)], ebw)

        def _grp(go, carry, first=(hh == 0)):
            for k in range(2):
                off = (go * 2 + k) * _LANES
                sv = ebs[pl.ds(off, _LANES)]
                dv = ebd[pl.ds(off, _LANES)]
                wv = ebw[pl.ds(off, _LANES)]
                key = lax.shift_right_logical(sv, 11)
                sloc = lax.bitwise_and(sv, jnp.int32(_BROWS - 1))
                dest = jnp.zeros((_LANES,), jnp.int32)
                nb = []
                for q in range(_NBKT):
                    m = key == q
                    mi = jnp.where(m, jnp.int32(1), jnp.int32(0))
                    pc = _prefix16(mi)
                    dest = jnp.where(
                        m, (s * _NBKT + q) * _SEG + carry[q] + pc - 1, dest)
                    nb.append(carry[q] + _bcast_lane_i(pc, _LANES - 1))
                carry = tuple(nb)

                def _drain():
                    pltpu.make_async_copy(
                        vs.at[k], ts_sh.at[dstg.at[k]], psem.at[k]).wait()
                    pltpu.make_async_copy(
                        vd.at[k], td_sh.at[dstg.at[k]], psem.at[k]).wait()
                    pltpu.make_async_copy(
                        vw.at[k], tw_sh.at[dstg.at[k]], psem.at[k]).wait()
                if first:
                    @pl.when(go > 0)
                    def _():
                        _drain()
                else:
                    _drain()
                dstg[k, pl.ds(0, _LANES)] = dest
                vs[k, pl.ds(0, _LANES)] = sloc
                vd[k, pl.ds(0, _LANES)] = dv
                vw[k, pl.ds(0, _LANES)] = wv
                pltpu.async_copy(vs.at[k], ts_sh.at[dstg.at[k]], psem.at[k])
                pltpu.async_copy(vd.at[k], td_sh.at[dstg.at[k]], psem.at[k])
                pltpu.async_copy(vw.at[k], tw_sh.at[dstg.at[k]], psem.at[k])
            return carry
        bases = lax.fori_loop(0, half // (2 * _LANES), _grp, bases)

    for k in range(2):
        pltpu.make_async_copy(vs.at[k], ts_sh.at[dstg.at[k]], psem.at[k]).wait()
        pltpu.make_async_copy(vd.at[k], td_sh.at[dstg.at[k]], psem.at[k]).wait()
        pltpu.make_async_copy(vw.at[k], tw_sh.at[dstg.at[k]], psem.at[k]).wait()

    pltpu.sync_copy(ts_sh.at[pl.ds(tbase, _TSEG)],
                    os_h.at[pl.ds(wid * _TSEG, _TSEG)])
    pltpu.sync_copy(td_sh.at[pl.ds(tbase, _TSEG)],
                    od_h.at[pl.ds(wid * _TSEG, _TSEG)])
    pltpu.sync_copy(tw_sh.at[pl.ds(tbase, _TSEG)],
                    ow_h.at[pl.ds(wid * _TSEG, _TSEG)])


def _make_presort():
    mesh = plsc.VectorSubcoreMesh(core_axis_name="c", subcore_axis_name="s")
    n = _NW * _TSEG
    return pl.kernel(
        _presort_body,
        out_type=(jax.ShapeDtypeStruct((n,), jnp.int32),
                  jax.ShapeDtypeStruct((n,), jnp.int32),
                  jax.ShapeDtypeStruct((n,), jnp.float32)),
        mesh=mesh,
        scratch_types=[
            pltpu.VMEM((_EPT // 2,), jnp.int32),
            pltpu.VMEM((_EPT // 2,), jnp.int32),
            pltpu.VMEM((_EPT // 2,), jnp.float32),
            pltpu.VMEM((2, _LANES), jnp.int32),
            pltpu.VMEM((2, _LANES), jnp.int32),
            pltpu.VMEM((2, _LANES), jnp.int32),
            pltpu.VMEM((2, _LANES), jnp.float32),
            pltpu.VMEM((_SEG,), jnp.int32),
            pltpu.VMEM((_SEG,), jnp.float32),
            pltpu.VMEM_SHARED((_NS * _TSEG,), jnp.int32),
            pltpu.VMEM_SHARED((_NS * _TSEG,), jnp.int32),
            pltpu.VMEM_SHARED((_NS * _TSEG,), jnp.float32),
            pltpu.SemaphoreType.DMA,
            pltpu.SemaphoreType.DMA((2,)),
        ],
    )


def _spmm_body(h_hbm, st, wt, dt, out_hbm,
               sring, wring, dring, rows, acc_sh, hstage,
               tsem, dsem, gsem, ssem, hsem):
    c = lax.axis_index("c")
    s = lax.axis_index("s")
    wid = c * _NS + s

    def _zrow(i, carry):
        for j in range(_NHID // _LANES):
            rows[0, i, pl.ds(j * _LANES, _LANES)] = jnp.zeros((_LANES,),
                                                              jnp.float32)
        return carry
    lax.fori_loop(0, _CHUNK, _zrow, 0)
    for q in range(_RPT // 40):
        pltpu.async_copy(rows.at[0, pl.ds(0, 40)],
                         acc_sh.at[pl.ds(s * _RPT + q * 40, 40)],
                         ssem.at[0])
    for q in range(_RPT // 40):
        pltpu.make_async_copy(rows.at[0, pl.ds(0, 40)],
                              acc_sh.at[pl.ds(s * _RPT, 40)],
                              ssem.at[0]).wait()
    plsc.subcore_barrier()

    def _scale_chunk(b):
        for g in range(_CHUNK // _LANES):
            w16 = wring[b, pl.ds(g * _LANES, _LANES)]
            for k in range(_LANES):
                wk = _bcast_lane(w16, k)
                e = g * _LANES + k
                for j in range(_NHID // _LANES):
                    sl = pl.ds(j * _LANES, _LANES)
                    rows[b, e, sl] = rows[b, e, sl] * wk

    for q in range(_NBKT):
        pltpu.async_copy(h_hbm.at[pl.ds(q * _BROWS + s * 128, 128)],
                         hstage.at[pl.ds(s * 128, 128)], hsem)
        pltpu.make_async_copy(h_hbm.at[pl.ds(q * _BROWS + s * 128, 128)],
                              hstage.at[pl.ds(s * 128, 128)], hsem).wait()
        plsc.subcore_barrier()

        for b in range(_NBUF):
            pltpu.async_copy(st.at[wid, q, b], sring.at[b], tsem.at[b])
            pltpu.async_copy(wt.at[wid, q, b], wring.at[b], tsem.at[b])
        for b in range(_NBUF - 1):
            pltpu.async_copy(dt.at[wid, q, b], dring.at[b], dsem.at[b])
        for b in range(_NBUF - 1):
            pltpu.make_async_copy(st.at[wid, q, b], sring.at[b],
                                  tsem.at[b]).wait()
            pltpu.make_async_copy(wt.at[wid, q, b], wring.at[b],
                                  tsem.at[b]).wait()
            pltpu.async_copy(hstage.at[sring.at[b]], rows.at[b], gsem.at[b])

        def _outer(to, carry):
            for b in range(_NBUF):
                t = to * _NBUF + b
                pb = (b - 1) % _NBUF
                pltpu.make_async_copy(hstage.at[sring.at[b]], rows.at[b],
                                      gsem.at[b]).wait()
                _scale_chunk(b)
                @pl.when(t + _NBUF < _BCAP)
                def _():
                    pltpu.async_copy(st.at[wid, q, t + _NBUF], sring.at[b],
                                     tsem.at[b])
                    pltpu.async_copy(wt.at[wid, q, t + _NBUF], wring.at[b],
                                     tsem.at[b])
                def _drain_prev():
                    pltpu.make_async_copy(rows.at[pb],
                                          acc_sh.at[dring.at[pb]],
                                          ssem.at[pb]).wait()
                if b == 0:
                    @pl.when(to > 0)
                    def _():
                        _drain_prev()
                else:
                    _drain_prev()
                @pl.when(t + _NBUF - 1 < _BCAP)
                def _():
                    pltpu.async_copy(dt.at[wid, q, t + _NBUF - 1],
                                     dring.at[pb], dsem.at[pb])
                pltpu.make_async_copy(dt.at[wid, q, t], dring.at[b],
                                      dsem.at[b]).wait()
                pltpu.async_copy(rows.at[b], acc_sh.at[dring.at[b]],
                                 ssem.at[b], add=True)
                @pl.when(t + _NBUF - 1 < _BCAP)
                def _():
                    pltpu.make_async_copy(st.at[wid, q, t + _NBUF - 1],
                                          sring.at[pb], tsem.at[pb]).wait()
                    pltpu.make_async_copy(wt.at[wid, q, t + _NBUF - 1],
                                          wring.at[pb], tsem.at[pb]).wait()
                    pltpu.async_copy(hstage.at[sring.at[pb]], rows.at[pb],
                                     gsem.at[pb])
            return carry
        lax.fori_loop(0, _BCAP // _NBUF, _outer, 0)

        pltpu.make_async_copy(rows.at[(_BCAP - 1) % _NBUF],
                              acc_sh.at[dring.at[(_BCAP - 1) % _NBUF]],
                              ssem.at[(_BCAP - 1) % _NBUF]).wait()
        plsc.subcore_barrier()

    pltpu.sync_copy(acc_sh.at[pl.ds(s * _RPT, _RPT)],
                    out_hbm.at[c, pl.ds(s * _RPT, _RPT)])


def _make_spmm():
    mesh = plsc.VectorSubcoreMesh(core_axis_name="c", subcore_axis_name="s")
    return pl.kernel(
        _spmm_body,
        out_type=jax.ShapeDtypeStruct((_NC, _NPAD, _NHID), jnp.float32),
        mesh=mesh,
        scratch_types=[
            pltpu.VMEM((_NBUF, _CHUNK), jnp.int32),
            pltpu.VMEM((_NBUF, _CHUNK), jnp.float32),
            pltpu.VMEM((_NBUF, _CHUNK), jnp.int32),
            pltpu.VMEM((_NBUF, _CHUNK, _NHID), jnp.float32),
            pltpu.VMEM_SHARED((_NPAD, _NHID), jnp.float32),
            pltpu.VMEM_SHARED((_BROWS, _NHID), jnp.float32),
            pltpu.SemaphoreType.DMA((_NBUF,)),
            pltpu.SemaphoreType.DMA((_NBUF,)),
            pltpu.SemaphoreType.DMA((_NBUF,)),
            pltpu.SemaphoreType.DMA((_NBUF,)),
            pltpu.SemaphoreType.DMA,
        ],
    )


_BN = 1024
_BNF = 1000


def _fc0_body(x_ref, w_ref, b_ref, o_ref):
    t = jnp.dot(x_ref[...], w_ref[...], preferred_element_type=jnp.float32)
    o_ref[...] = jnp.maximum(t + b_ref[...], 0.0)


def _dense_body(theta, p_ref, h0_ref, h_ref, w_ref, o_ref):
    sup = (1.0 - _ALPHA) * (p_ref[0] + p_ref[1]) + _ALPHA * h0_ref[...]
    t = jnp.dot(sup, w_ref[...], preferred_element_type=jnp.float32)
    o_ref[...] = jnp.maximum(theta * t + (1.0 - theta) * sup + h_ref[...], 0.0)


def _final_body(h_ref, w_ref, b_ref, o_ref):
    t = jnp.dot(h_ref[...], w_ref[...], preferred_element_type=jnp.float32)
    o_ref[...] = jax.nn.sigmoid(t + b_ref[...])


def _dense_final_body(theta, p_ref, h0_ref, h_ref, w_ref, wo_ref, bo_ref,
                      o_ref):
    sup = (1.0 - _ALPHA) * (p_ref[0] + p_ref[1]) + _ALPHA * h0_ref[...]
    t = jnp.dot(sup, w_ref[...], preferred_element_type=jnp.float32)
    hn = jnp.maximum(theta * t + (1.0 - theta) * sup + h_ref[...], 0.0)
    t2 = jnp.dot(hn, wo_ref[...], preferred_element_type=jnp.float32)
    o_ref[...] = jax.nn.sigmoid(t2 + bo_ref[...])


def _dense_final(p, h0, h, W, theta, Wout, bout):
    return pl.pallas_call(
        functools.partial(_dense_final_body, theta),
        grid=(_N // _BNF,),
        in_specs=[
            pl.BlockSpec((_NC, _BNF, _NHID), lambda i: (0, i, 0)),
            pl.BlockSpec((_BNF, _NHID), lambda i: (i, 0)),
            pl.BlockSpec((_BNF, _NHID), lambda i: (i, 0)),
            pl.BlockSpec((_NHID, _NHID), lambda i: (0, 0)),
            pl.BlockSpec((_NHID, _NCLASS), lambda i: (0, 0)),
            pl.BlockSpec((1, _NCLASS), lambda i: (0, 0)),
        ],
        out_specs=pl.BlockSpec((_BNF, _NCLASS), lambda i: (i, 0)),
        out_shape=jax.ShapeDtypeStruct((_N, _NCLASS), jnp.float32),
    )(p, h0, h, W, Wout, bout.reshape(1, _NCLASS))


def _fc0(x, W0, b0):
    return pl.pallas_call(
        _fc0_body,
        grid=(_NPAD // _BN,),
        in_specs=[
            pl.BlockSpec((_BN, _NFEAT), lambda i: (i, 0)),
            pl.BlockSpec((_NFEAT, _NHID), lambda i: (0, 0)),
            pl.BlockSpec((1, _NHID), lambda i: (0, 0)),
        ],
        out_specs=pl.BlockSpec((_BN, _NHID), lambda i: (i, 0)),
        out_shape=jax.ShapeDtypeStruct((_NPAD, _NHID), jnp.float32),
    )(x, W0, b0.reshape(1, _NHID))


def _dense(p, h0, h, W, theta):
    return pl.pallas_call(
        functools.partial(_dense_body, theta),
        grid=(_NPAD // _BN,),
        in_specs=[
            pl.BlockSpec((_NC, _BN, _NHID), lambda i: (0, i, 0)),
            pl.BlockSpec((_BN, _NHID), lambda i: (i, 0)),
            pl.BlockSpec((_BN, _NHID), lambda i: (i, 0)),
            pl.BlockSpec((_NHID, _NHID), lambda i: (0, 0)),
        ],
        out_specs=pl.BlockSpec((_BN, _NHID), lambda i: (i, 0)),
        out_shape=jax.ShapeDtypeStruct((_NPAD, _NHID), jnp.float32),
    )(p, h0, h, W)


def _final(h, Wout, bout):
    return pl.pallas_call(
        _final_body,
        grid=(_N // _BNF,),
        in_specs=[
            pl.BlockSpec((_BNF, _NHID), lambda i: (i, 0)),
            pl.BlockSpec((_NHID, _NCLASS), lambda i: (0, 0)),
            pl.BlockSpec((1, _NCLASS), lambda i: (0, 0)),
        ],
        out_specs=pl.BlockSpec((_BNF, _NCLASS), lambda i: (i, 0)),
        out_shape=jax.ShapeDtypeStruct((_N, _NCLASS), jnp.float32),
    )(h, Wout, bout.reshape(1, _NCLASS))


def kernel(x, edge_index, edge_weight, W0, b0, Wc, Wout, bout):
    ppt = _EPT - _E // _NW
    sp = jnp.concatenate(
        [edge_index[0].reshape(_NW, _E // _NW),
         jnp.full((_NW, ppt), _NPAD - 1, jnp.int32)], axis=1)
    dp = jnp.concatenate(
        [edge_index[1].reshape(_NW, _E // _NW),
         jnp.zeros((_NW, ppt), jnp.int32)], axis=1)
    wpf = jnp.concatenate(
        [edge_weight.reshape(_NW, _E // _NW),
         jnp.zeros((_NW, ppt), jnp.float32)], axis=1)
    st_f, dt_f, wt_f = _make_presort()(sp, dp, wpf)
    st = st_f.reshape(_NW, _NBKT, _BCAP, _CHUNK)
    dt = dt_f.reshape(_NW, _NBKT, _BCAP, _CHUNK)
    wt = wt_f.reshape(_NW, _NBKT, _BCAP, _CHUNK)

    xpad = jnp.concatenate(
        [x, jnp.zeros((_NPAD - _N, _NFEAT), jnp.float32)], axis=0)
    spmm = _make_spmm()
    h = _fc0(xpad, W0, b0)
    h0 = h
    for i in range(_NLAYERS - 1):
        theta = math.log(_LAMDA / (i + 1) + 1.0)
        p = spmm(h, st, wt, dt)
        h = _dense(p, h0, h, Wc[i], theta)
    theta = math.log(_LAMDA / _NLAYERS + 1.0)
    p = spmm(h, st, wt, dt)
    return _dense_final(p, h0, h, Wc[_NLAYERS - 1], theta, Wout, bout)


# speedup vs baseline: 1.0348x; 1.0006x over previous
import functools
import math

import jax
import jax.numpy as jnp
from jax import lax
from jax.experimental import pallas as pl
from jax.experimental.pallas import tpu as pltpu
from jax.experimental.pallas import tpu_sc as plsc

_N = 10000
_E = 320000
_NFEAT = 128
_NHID = 128
_NCLASS = 64
_NLAYERS = 4
_LAMDA = 0.5
_ALPHA = 0.1

_NC = 2
_NS = 16
_NW = _NC * _NS
_NPAD = 10240
_EPT = _NPAD
_NBKT = 5
_BROWS = 2048
_CHUNK = 32
_BCAP = 72
_SEG = _BCAP * _CHUNK
_TSEG = _NBKT * _SEG
_NBUF = 3
_LANES = 16
_RPT = _NPAD // _NS


def _bcast_lane(v16, k):
    return lax.gather(
        v16, jnp.full((_LANES, 1), k, jnp.int32),
        lax.GatherDimensionNumbers(
            offset_dims=(), collapsed_slice_dims=(0,), start_index_map=(0,)),
        slice_sizes=(1,),
        mode=lax.GatherScatterMode.PROMISE_IN_BOUNDS)


_bcast_lane_i = _bcast_lane


def _gather16(v, idx_const):
    return lax.gather(
        v, idx_const[:, None],
        lax.GatherDimensionNumbers(
            offset_dims=(), collapsed_slice_dims=(0,), start_index_map=(0,)),
        slice_sizes=(1,),
        mode=lax.GatherScatterMode.PROMISE_IN_BOUNDS)


def _prefix16(mi):
    lane = jnp.arange(_LANES, dtype=jnp.int32)
    pc = mi
    for k in (1, 2, 4, 8):
        sh = _gather16(pc, jnp.maximum(lane - k, 0))
        pc = pc + jnp.where(lane >= k, sh, 0)
    return pc


def _presort_body(sp, dp, wpf, os_h, od_h, ow_h,
                  ebs, ebd, ebw, dstg, vs, vd, vw, zbi, zbf,
                  ts_sh, td_sh, tw_sh, lsem, psem):
    c = lax.axis_index("c")
    s = lax.axis_index("s")
    wid = c * _NS + s
    tbase = s * _TSEG

    def _z(i, cr):
        zbi[pl.ds(i * _LANES, _LANES)] = jnp.zeros((_LANES,), jnp.int32)
        zbf[pl.ds(i * _LANES, _LANES)] = jnp.zeros((_LANES,), jnp.float32)
        return cr
    lax.fori_loop(0, _SEG // _LANES, _z, 0)
    for q in range(_NBKT):
        pltpu.async_copy(zbi, ts_sh.at[pl.ds(tbase + q * _SEG, _SEG)], lsem)
        pltpu.async_copy(zbi, td_sh.at[pl.ds(tbase + q * _SEG, _SEG)], lsem)
        pltpu.async_copy(zbf, tw_sh.at[pl.ds(tbase + q * _SEG, _SEG)], lsem)
    for q in range(_NBKT):
        for _ in range(3):
            pltpu.make_async_copy(
                zbi, ts_sh.at[pl.ds(tbase, _SEG)], lsem).wait()

    bases = (jnp.zeros((_LANES,), jnp.int32),) * _NBKT
    half = _EPT // 2
    for hh in range(2):
        pltpu.sync_copy(sp.at[wid, pl.ds(hh * half, half)], ebs)
        pltpu.sync_copy(dp.at[wid, pl.ds(hh * half, half)], ebd)
        pltpu.sync_copy(wpf.at[wid, pl.ds(hh * half, half)], ebw)

        def _grp(go, carry, first=(hh == 0)):
            for k in range(2):
                off = (go * 2 + k) * _LANES
                sv = ebs[pl.ds(off, _LANES)]
                dv = ebd[pl.ds(off, _LANES)]
                wv = ebw[pl.ds(off, _LANES)]
                key = lax.shift_right_logical(sv, 11)
                sloc = lax.bitwise_and(sv, jnp.int32(_BROWS - 1))
                dest = jnp.zeros((_LANES,), jnp.int32)
                nb = []
                for q in range(_NBKT):
                    m = key == q
                    mi = jnp.where(m, jnp.int32(1), jnp.int32(0))
                    pc = _prefix16(mi)
                    dest = jnp.where(
                        m, (s * _NBKT + q) * _SEG + carry[q] + pc - 1, dest)
                    nb.append(carry[q] + _bcast_lane_i(pc, _LANES - 1))
                carry = tuple(nb)

                def _drain():
                    pltpu.make_async_copy(
                        vs.at[k], ts_sh.at[dstg.at[k]], psem.at[k]).wait()
                    pltpu.make_async_copy(
                        vd.at[k], td_sh.at[dstg.at[k]], psem.at[k]).wait()
                    pltpu.make_async_copy(
                        vw.at[k], tw_sh.at[dstg.at[k]], psem.at[k]).wait()
                if first:
                    @pl.when(go > 0)
                    def _():
                        _drain()
                else:
                    _drain()
                dstg[k, pl.ds(0, _LANES)] = dest
                vs[k, pl.ds(0, _LANES)] = sloc
                vd[k, pl.ds(0, _LANES)] = dv
                vw[k, pl.ds(0, _LANES)] = wv
                pltpu.async_copy(vs.at[k], ts_sh.at[dstg.at[k]], psem.at[k])
                pltpu.async_copy(vd.at[k], td_sh.at[dstg.at[k]], psem.at[k])
                pltpu.async_copy(vw.at[k], tw_sh.at[dstg.at[k]], psem.at[k])
            return carry
        bases = lax.fori_loop(0, half // (2 * _LANES), _grp, bases)

    for k in range(2):
        pltpu.make_async_copy(vs.at[k], ts_sh.at[dstg.at[k]], psem.at[k]).wait()
        pltpu.make_async_copy(vd.at[k], td_sh.at[dstg.at[k]], psem.at[k]).wait()
        pltpu.make_async_copy(vw.at[k], tw_sh.at[dstg.at[k]], psem.at[k]).wait()

    pltpu.sync_copy(ts_sh.at[pl.ds(tbase, _TSEG)],
                    os_h.at[pl.ds(wid * _TSEG, _TSEG)])
    pltpu.sync_copy(td_sh.at[pl.ds(tbase, _TSEG)],
                    od_h.at[pl.ds(wid * _TSEG, _TSEG)])
    pltpu.sync_copy(tw_sh.at[pl.ds(tbase, _TSEG)],
                    ow_h.at[pl.ds(wid * _TSEG, _TSEG)])


def _make_presort():
    mesh = plsc.VectorSubcoreMesh(core_axis_name="c", subcore_axis_name="s")
    n = _NW * _TSEG
    return pl.kernel(
        _presort_body,
        out_type=(jax.ShapeDtypeStruct((n,), jnp.int32),
                  jax.ShapeDtypeStruct((n,), jnp.int32),
                  jax.ShapeDtypeStruct((n,), jnp.float32)),
        mesh=mesh,
        scratch_types=[
            pltpu.VMEM((_EPT // 2,), jnp.int32),
            pltpu.VMEM((_EPT // 2,), jnp.int32),
            pltpu.VMEM((_EPT // 2,), jnp.float32),
            pltpu.VMEM((2, _LANES), jnp.int32),
            pltpu.VMEM((2, _LANES), jnp.int32),
            pltpu.VMEM((2, _LANES), jnp.int32),
            pltpu.VMEM((2, _LANES), jnp.float32),
            pltpu.VMEM((_SEG,), jnp.int32),
            pltpu.VMEM((_SEG,), jnp.float32),
            pltpu.VMEM_SHARED((_NS * _TSEG,), jnp.int32),
            pltpu.VMEM_SHARED((_NS * _TSEG,), jnp.int32),
            pltpu.VMEM_SHARED((_NS * _TSEG,), jnp.float32),
            pltpu.SemaphoreType.DMA,
            pltpu.SemaphoreType.DMA((2,)),
        ],
    )


def _spmm_body(h_hbm, st, wt, dt, out_hbm,
               sring, wring, dring, rows, acc_sh, hstage,
               tsem, dsem, gsem, ssem, hsem):
    c = lax.axis_index("c")
    s = lax.axis_index("s")
    wid = c * _NS + s

    def _zrow(i, carry):
        for j in range(_NHID // _LANES):
            rows[0, i, pl.ds(j * _LANES, _LANES)] = jnp.zeros((_LANES,),
                                                              jnp.float32)
        return carry
    lax.fori_loop(0, _CHUNK, _zrow, 0)
    for q in range(_RPT // 40):
        pltpu.async_copy(rows.at[0, pl.ds(0, 40)],
                         acc_sh.at[pl.ds(s * _RPT + q * 40, 40)],
                         ssem.at[0])
    for q in range(_RPT // 40):
        pltpu.make_async_copy(rows.at[0, pl.ds(0, 40)],
                              acc_sh.at[pl.ds(s * _RPT, 40)],
                              ssem.at[0]).wait()
    plsc.subcore_barrier()

    def _scale_chunk(b):
        for g in range(_CHUNK // _LANES):
            w16 = wring[b, pl.ds(g * _LANES, _LANES)]
            for k in range(_LANES):
                wk = _bcast_lane(w16, k)
                e = g * _LANES + k
                for j in range(_NHID // _LANES):
                    sl = pl.ds(j * _LANES, _LANES)
                    rows[b, e, sl] = rows[b, e, sl] * wk

    for q in range(_NBKT):
        pltpu.async_copy(h_hbm.at[pl.ds(q * _BROWS + s * 128, 128)],
                         hstage.at[pl.ds(s * 128, 128)], hsem)
        pltpu.make_async_copy(h_hbm.at[pl.ds(q * _BROWS + s * 128, 128)],
                              hstage.at[pl.ds(s * 128, 128)], hsem).wait()
        plsc.subcore_barrier()

        for b in range(_NBUF):
            pltpu.async_copy(st.at[wid, q, b], sring.at[b], tsem.at[b])
            pltpu.async_copy(wt.at[wid, q, b], wring.at[b], tsem.at[b])
        for b in range(_NBUF - 1):
            pltpu.async_copy(dt.at[wid, q, b], dring.at[b], dsem.at[b])
        for b in range(_NBUF - 1):
            pltpu.make_async_copy(st.at[wid, q, pl.ds(0, 2)],
                                  sring.at[pl.ds(0, 2)], tsem.at[b]).wait()
            pltpu.async_copy(hstage.at[sring.at[b]], rows.at[b], gsem.at[b])

        def _outer(to, carry):
            for b in range(_NBUF):
                t = to * _NBUF + b
                pb = (b - 1) % _NBUF
                pltpu.make_async_copy(hstage.at[sring.at[b]], rows.at[b],
                                      gsem.at[b]).wait()
                _scale_chunk(b)
                @pl.when(t + _NBUF < _BCAP)
                def _():
                    pltpu.async_copy(st.at[wid, q, t + _NBUF], sring.at[b],
                                     tsem.at[b])
                    pltpu.async_copy(wt.at[wid, q, t + _NBUF], wring.at[b],
                                     tsem.at[b])
                def _drain_prev():
                    pltpu.make_async_copy(rows.at[pb],
                                          acc_sh.at[dring.at[pb]],
                                          ssem.at[pb]).wait()
                if b == 0:
                    @pl.when(to > 0)
                    def _():
                        _drain_prev()
                else:
                    _drain_prev()
                @pl.when(t + _NBUF - 1 < _BCAP)
                def _():
                    pltpu.async_copy(dt.at[wid, q, t + _NBUF - 1],
                                     dring.at[pb], dsem.at[pb])
                pltpu.make_async_copy(dt.at[wid, q, t], dring.at[b],
                                      dsem.at[b]).wait()
                pltpu.async_copy(rows.at[b], acc_sh.at[dring.at[b]],
                                 ssem.at[b], add=True)
                @pl.when(t + _NBUF - 1 < _BCAP)
                def _():
                    pltpu.make_async_copy(st.at[wid, q, pl.ds(0, 2)],
                                          sring.at[pl.ds(0, 2)],
                                          tsem.at[pb]).wait()
                    pltpu.async_copy(hstage.at[sring.at[pb]], rows.at[pb],
                                     gsem.at[pb])
            return carry
        lax.fori_loop(0, _BCAP // _NBUF, _outer, 0)

        pltpu.make_async_copy(rows.at[(_BCAP - 1) % _NBUF],
                              acc_sh.at[dring.at[(_BCAP - 1) % _NBUF]],
                              ssem.at[(_BCAP - 1) % _NBUF]).wait()
        plsc.subcore_barrier()

    pltpu.sync_copy(acc_sh.at[pl.ds(s * _RPT, _RPT)],
                    out_hbm.at[c, pl.ds(s * _RPT, _RPT)])


def _make_spmm():
    mesh = plsc.VectorSubcoreMesh(core_axis_name="c", subcore_axis_name="s")
    return pl.kernel(
        _spmm_body,
        out_type=jax.ShapeDtypeStruct((_NC, _NPAD, _NHID), jnp.float32),
        mesh=mesh,
        scratch_types=[
            pltpu.VMEM((_NBUF, _CHUNK), jnp.int32),
            pltpu.VMEM((_NBUF, _CHUNK), jnp.float32),
            pltpu.VMEM((_NBUF, _CHUNK), jnp.int32),
            pltpu.VMEM((_NBUF, _CHUNK, _NHID), jnp.float32),
            pltpu.VMEM_SHARED((_NPAD, _NHID), jnp.float32),
            pltpu.VMEM_SHARED((_BROWS, _NHID), jnp.float32),
            pltpu.SemaphoreType.DMA((_NBUF,)),
            pltpu.SemaphoreType.DMA((_NBUF,)),
            pltpu.SemaphoreType.DMA((_NBUF,)),
            pltpu.SemaphoreType.DMA((_NBUF,)),
            pltpu.SemaphoreType.DMA,
        ],
    )


_BN = 1024
_BNF = 1000


def _fc0_body(x_ref, w_ref, b_ref, o_ref):
    t = jnp.dot(x_ref[...], w_ref[...], preferred_element_type=jnp.float32)
    o_ref[...] = jnp.maximum(t + b_ref[...], 0.0)


def _dense_body(theta, p_ref, h0_ref, h_ref, w_ref, o_ref):
    sup = (1.0 - _ALPHA) * (p_ref[0] + p_ref[1]) + _ALPHA * h0_ref[...]
    t = jnp.dot(sup, w_ref[...], preferred_element_type=jnp.float32)
    o_ref[...] = jnp.maximum(theta * t + (1.0 - theta) * sup + h_ref[...], 0.0)


def _final_body(h_ref, w_ref, b_ref, o_ref):
    t = jnp.dot(h_ref[...], w_ref[...], preferred_element_type=jnp.float32)
    o_ref[...] = jax.nn.sigmoid(t + b_ref[...])


def _dense_final_body(theta, p_ref, h0_ref, h_ref, w_ref, wo_ref, bo_ref,
                      o_ref):
    sup = (1.0 - _ALPHA) * (p_ref[0] + p_ref[1]) + _ALPHA * h0_ref[...]
    t = jnp.dot(sup, w_ref[...], preferred_element_type=jnp.float32)
    hn = jnp.maximum(theta * t + (1.0 - theta) * sup + h_ref[...], 0.0)
    t2 = jnp.dot(hn, wo_ref[...], preferred_element_type=jnp.float32)
    o_ref[...] = jax.nn.sigmoid(t2 + bo_ref[...])


def _dense_final(p, h0, h, W, theta, Wout, bout):
    return pl.pallas_call(
        functools.partial(_dense_final_body, theta),
        grid=(_N // _BNF,),
        in_specs=[
            pl.BlockSpec((_NC, _BNF, _NHID), lambda i: (0, i, 0)),
            pl.BlockSpec((_BNF, _NHID), lambda i: (i, 0)),
            pl.BlockSpec((_BNF, _NHID), lambda i: (i, 0)),
            pl.BlockSpec((_NHID, _NHID), lambda i: (0, 0)),
            pl.BlockSpec((_NHID, _NCLASS), lambda i: (0, 0)),
            pl.BlockSpec((1, _NCLASS), lambda i: (0, 0)),
        ],
        out_specs=pl.BlockSpec((_BNF, _NCLASS), lambda i: (i, 0)),
        out_shape=jax.ShapeDtypeStruct((_N, _NCLASS), jnp.float32),
    )(p, h0, h, W, Wout, bout.reshape(1, _NCLASS))


def _fc0(x, W0, b0):
    return pl.pallas_call(
        _fc0_body,
        grid=(_NPAD // _BN,),
        in_specs=[
            pl.BlockSpec((_BN, _NFEAT), lambda i: (i, 0)),
            pl.BlockSpec((_NFEAT, _NHID), lambda i: (0, 0)),
            pl.BlockSpec((1, _NHID), lambda i: (0, 0)),
        ],
        out_specs=pl.BlockSpec((_BN, _NHID), lambda i: (i, 0)),
        out_shape=jax.ShapeDtypeStruct((_NPAD, _NHID), jnp.float32),
    )(x, W0, b0.reshape(1, _NHID))


def _dense(p, h0, h, W, theta):
    return pl.pallas_call(
        functools.partial(_dense_body, theta),
        grid=(_NPAD // _BN,),
        in_specs=[
            pl.BlockSpec((_NC, _BN, _NHID), lambda i: (0, i, 0)),
            pl.BlockSpec((_BN, _NHID), lambda i: (i, 0)),
            pl.BlockSpec((_BN, _NHID), lambda i: (i, 0)),
            pl.BlockSpec((_NHID, _NHID), lambda i: (0, 0)),
        ],
        out_specs=pl.BlockSpec((_BN, _NHID), lambda i: (i, 0)),
        out_shape=jax.ShapeDtypeStruct((_NPAD, _NHID), jnp.float32),
    )(p, h0, h, W)


def _final(h, Wout, bout):
    return pl.pallas_call(
        _final_body,
        grid=(_N // _BNF,),
        in_specs=[
            pl.BlockSpec((_BNF, _NHID), lambda i: (i, 0)),
            pl.BlockSpec((_NHID, _NCLASS), lambda i: (0, 0)),
            pl.BlockSpec((1, _NCLASS), lambda i: (0, 0)),
        ],
        out_specs=pl.BlockSpec((_BNF, _NCLASS), lambda i: (i, 0)),
        out_shape=jax.ShapeDtypeStruct((_N, _NCLASS), jnp.float32),
    )(h, Wout, bout.reshape(1, _NCLASS))


def kernel(x, edge_index, edge_weight, W0, b0, Wc, Wout, bout):
    ppt = _EPT - _E // _NW
    sp = jnp.concatenate(
        [edge_index[0].reshape(_NW, _E // _NW),
         jnp.full((_NW, ppt), _NPAD - 1, jnp.int32)], axis=1)
    dp = jnp.concatenate(
        [edge_index[1].reshape(_NW, _E // _NW),
         jnp.zeros((_NW, ppt), jnp.int32)], axis=1)
    wpf = jnp.concatenate(
        [edge_weight.reshape(_NW, _E // _NW),
         jnp.zeros((_NW, ppt), jnp.float32)], axis=1)
    st_f, dt_f, wt_f = _make_presort()(sp, dp, wpf)
    st = st_f.reshape(_NW, _NBKT, _BCAP, _CHUNK)
    dt = dt_f.reshape(_NW, _NBKT, _BCAP, _CHUNK)
    wt = wt_f.reshape(_NW, _NBKT, _BCAP, _CHUNK)

    xpad = jnp.concatenate(
        [x, jnp.zeros((_NPAD - _N, _NFEAT), jnp.float32)], axis=0)
    spmm = _make_spmm()
    h = _fc0(xpad, W0, b0)
    h0 = h
    for i in range(_NLAYERS - 1):
        theta = math.log(_LAMDA / (i + 1) + 1.0)
        p = spmm(h, st, wt, dt)
        h = _dense(p, h0, h, Wc[i], theta)
    theta = math.log(_LAMDA / _NLAYERS + 1.0)
    p = spmm(h, st, wt, dt)
    return _dense_final(p, h0, h, Wc[_NLAYERS - 1], theta, Wout, bout)


# speedup vs baseline: 1.0754x; 1.0392x over previous
import functools
import math

import jax
import jax.numpy as jnp
from jax import lax
from jax.experimental import pallas as pl
from jax.experimental.pallas import tpu as pltpu
from jax.experimental.pallas import tpu_sc as plsc

_N = 10000
_E = 320000
_NFEAT = 128
_NHID = 128
_NCLASS = 64
_NLAYERS = 4
_LAMDA = 0.5
_ALPHA = 0.1

_NC = 2
_NS = 16
_NW = _NC * _NS
_NPAD = 10240
_EPT = _NPAD
_NBKT = 5
_BROWS = 2048
_CHUNK = 32
_BCAP = 72
_SEG = _BCAP * _CHUNK
_TSEG = _NBKT * _SEG
_NBUF = 3
_LANES = 16
_RPT = _NPAD // _NS


def _bcast_lane(v16, k):
    return lax.gather(
        v16, jnp.full((_LANES, 1), k, jnp.int32),
        lax.GatherDimensionNumbers(
            offset_dims=(), collapsed_slice_dims=(0,), start_index_map=(0,)),
        slice_sizes=(1,),
        mode=lax.GatherScatterMode.PROMISE_IN_BOUNDS)


_bcast_lane_i = _bcast_lane


def _gather16(v, idx_const):
    return lax.gather(
        v, idx_const[:, None],
        lax.GatherDimensionNumbers(
            offset_dims=(), collapsed_slice_dims=(0,), start_index_map=(0,)),
        slice_sizes=(1,),
        mode=lax.GatherScatterMode.PROMISE_IN_BOUNDS)


def _prefix16(mi):
    lane = jnp.arange(_LANES, dtype=jnp.int32)
    pc = mi
    for k in (1, 2, 4, 8):
        sh = _gather16(pc, jnp.maximum(lane - k, 0))
        pc = pc + jnp.where(lane >= k, sh, 0)
    return pc


def _presort_body(sp, dp, wpf, os_h, od_h, ow_h,
                  ebs, ebd, ebw, dstg, vs, vd, vw, zbi, zbf,
                  ts_sh, td_sh, tw_sh, lsem, psem):
    c = lax.axis_index("c")
    s = lax.axis_index("s")
    wid = c * _NS + s
    tbase = s * _TSEG

    def _z(i, cr):
        zbi[pl.ds(i * _LANES, _LANES)] = jnp.zeros((_LANES,), jnp.int32)
        zbf[pl.ds(i * _LANES, _LANES)] = jnp.zeros((_LANES,), jnp.float32)
        return cr
    lax.fori_loop(0, _SEG // _LANES, _z, 0)
    for q in range(_NBKT):
        pltpu.async_copy(zbi, ts_sh.at[pl.ds(tbase + q * _SEG, _SEG)], lsem)
        pltpu.async_copy(zbi, td_sh.at[pl.ds(tbase + q * _SEG, _SEG)], lsem)
        pltpu.async_copy(zbf, tw_sh.at[pl.ds(tbase + q * _SEG, _SEG)], lsem)
    for q in range(_NBKT):
        for _ in range(3):
            pltpu.make_async_copy(
                zbi, ts_sh.at[pl.ds(tbase, _SEG)], lsem).wait()

    bases = (jnp.zeros((_LANES,), jnp.int32),) * _NBKT
    half = _EPT // 2
    for hh in range(2):
        pltpu.sync_copy(sp.at[wid, pl.ds(hh * half, half)], ebs)
        pltpu.sync_copy(dp.at[wid, pl.ds(hh * half, half)], ebd)
        pltpu.sync_copy(wpf.at[wid, pl.ds(hh * half, half)], ebw)

        def _grp(go, carry, first=(hh == 0)):
            for k in range(2):
                off = (go * 2 + k) * _LANES
                sv = ebs[pl.ds(off, _LANES)]
                dv = ebd[pl.ds(off, _LANES)]
                wv = ebw[pl.ds(off, _LANES)]
                key = lax.shift_right_logical(sv, 11)
                sloc = lax.bitwise_and(sv, jnp.int32(_BROWS - 1))
                dest = jnp.zeros((_LANES,), jnp.int32)
                nb = []
                for q in range(_NBKT):
                    m = key == q
                    mi = jnp.where(m, jnp.int32(1), jnp.int32(0))
                    pc = _prefix16(mi)
                    dest = jnp.where(
                        m, (s * _NBKT + q) * _SEG + carry[q] + pc - 1, dest)
                    nb.append(carry[q] + _bcast_lane_i(pc, _LANES - 1))
                carry = tuple(nb)

                def _drain():
                    pltpu.make_async_copy(
                        vs.at[k], ts_sh.at[dstg.at[k]], psem.at[k]).wait()
                    pltpu.make_async_copy(
                        vd.at[k], td_sh.at[dstg.at[k]], psem.at[k]).wait()
                    pltpu.make_async_copy(
                        vw.at[k], tw_sh.at[dstg.at[k]], psem.at[k]).wait()
                if first:
                    @pl.when(go > 0)
                    def _():
                        _drain()
                else:
                    _drain()
                dstg[k, pl.ds(0, _LANES)] = dest
                vs[k, pl.ds(0, _LANES)] = sloc
                vd[k, pl.ds(0, _LANES)] = dv
                vw[k, pl.ds(0, _LANES)] = wv
                pltpu.async_copy(vs.at[k], ts_sh.at[dstg.at[k]], psem.at[k])
                pltpu.async_copy(vd.at[k], td_sh.at[dstg.at[k]], psem.at[k])
                pltpu.async_copy(vw.at[k], tw_sh.at[dstg.at[k]], psem.at[k])
            return carry
        bases = lax.fori_loop(0, half // (2 * _LANES), _grp, bases)

    for k in range(2):
        pltpu.make_async_copy(vs.at[k], ts_sh.at[dstg.at[k]], psem.at[k]).wait()
        pltpu.make_async_copy(vd.at[k], td_sh.at[dstg.at[k]], psem.at[k]).wait()
        pltpu.make_async_copy(vw.at[k], tw_sh.at[dstg.at[k]], psem.at[k]).wait()

    pltpu.sync_copy(ts_sh.at[pl.ds(tbase, _TSEG)],
                    os_h.at[pl.ds(wid * _TSEG, _TSEG)])
    pltpu.sync_copy(td_sh.at[pl.ds(tbase, _TSEG)],
                    od_h.at[pl.ds(wid * _TSEG, _TSEG)])
    pltpu.sync_copy(tw_sh.at[pl.ds(tbase, _TSEG)],
                    ow_h.at[pl.ds(wid * _TSEG, _TSEG)])


def _make_presort():
    mesh = plsc.VectorSubcoreMesh(core_axis_name="c", subcore_axis_name="s")
    n = _NW * _TSEG
    return pl.kernel(
        _presort_body,
        out_type=(jax.ShapeDtypeStruct((n,), jnp.int32),
                  jax.ShapeDtypeStruct((n,), jnp.int32),
                  jax.ShapeDtypeStruct((n,), jnp.float32)),
        mesh=mesh,
        scratch_types=[
            pltpu.VMEM((_EPT // 2,), jnp.int32),
            pltpu.VMEM((_EPT // 2,), jnp.int32),
            pltpu.VMEM((_EPT // 2,), jnp.float32),
            pltpu.VMEM((2, _LANES), jnp.int32),
            pltpu.VMEM((2, _LANES), jnp.int32),
            pltpu.VMEM((2, _LANES), jnp.int32),
            pltpu.VMEM((2, _LANES), jnp.float32),
            pltpu.VMEM((_SEG,), jnp.int32),
            pltpu.VMEM((_SEG,), jnp.float32),
            pltpu.VMEM_SHARED((_NS * _TSEG,), jnp.int32),
            pltpu.VMEM_SHARED((_NS * _TSEG,), jnp.int32),
            pltpu.VMEM_SHARED((_NS * _TSEG,), jnp.float32),
            pltpu.SemaphoreType.DMA,
            pltpu.SemaphoreType.DMA((2,)),
        ],
    )


def _spmm_body(h_hbm, st, wt, dt, out_hbm,
               sring, wring, dring, rows, acc_sh, hstage,
               tsem, dsem, gsem, ssem, hsem):
    c = lax.axis_index("c")
    s = lax.axis_index("s")
    wid = c * _NS + s

    def _zrow(i, carry):
        for j in range(_NHID // _LANES):
            rows[0, i, pl.ds(j * _LANES, _LANES)] = jnp.zeros((_LANES,),
                                                              jnp.float32)
        return carry
    lax.fori_loop(0, _CHUNK, _zrow, 0)
    for q in range(_RPT // 40):
        pltpu.async_copy(rows.at[0, pl.ds(0, 40)],
                         acc_sh.at[pl.ds(s * _RPT + q * 40, 40)],
                         ssem.at[0])
    for q in range(_RPT // 40):
        pltpu.make_async_copy(rows.at[0, pl.ds(0, 40)],
                              acc_sh.at[pl.ds(s * _RPT, 40)],
                              ssem.at[0]).wait()
    plsc.subcore_barrier()

    def _scale_chunk(b):
        for g in range(_CHUNK // _LANES):
            w16 = wring[b, pl.ds(g * _LANES, _LANES)]
            for k in range(_LANES):
                wk = _bcast_lane(w16, k)
                e = g * _LANES + k
                for j in range(_NHID // _LANES):
                    sl = pl.ds(j * _LANES, _LANES)
                    rows[b, e, sl] = rows[b, e, sl] * wk

    for q in range(_NBKT):
        pltpu.async_copy(h_hbm.at[pl.ds(q * _BROWS + s * 128, 128)],
                         hstage.at[pl.ds(s * 128, 128)], hsem)
        pltpu.make_async_copy(h_hbm.at[pl.ds(q * _BROWS + s * 128, 128)],
                              hstage.at[pl.ds(s * 128, 128)], hsem).wait()
        plsc.subcore_barrier()

        for b in range(_NBUF):
            pltpu.async_copy(st.at[wid, q, b], sring.at[b], tsem.at[b])
            pltpu.async_copy(wt.at[wid, q, b], wring.at[b], tsem.at[b])
        for b in range(_NBUF - 1):
            pltpu.async_copy(dt.at[wid, q, b], dring.at[b], dsem.at[b])
        for b in range(_NBUF - 1):
            pltpu.make_async_copy(st.at[wid, q, pl.ds(0, 2)],
                                  sring.at[pl.ds(0, 2)], tsem.at[b]).wait()
            pltpu.async_copy(hstage.at[sring.at[b]], rows.at[b], gsem.at[b])

        def _outer(to, carry):
            for b in range(_NBUF):
                t = to * _NBUF + b
                pb = (b - 1) % _NBUF
                pltpu.make_async_copy(hstage.at[sring.at[b]], rows.at[b],
                                      gsem.at[b]).wait()
                _scale_chunk(b)
                @pl.when(t + _NBUF < _BCAP)
                def _():
                    pltpu.async_copy(st.at[wid, q, t + _NBUF], sring.at[b],
                                     tsem.at[b])
                    pltpu.async_copy(wt.at[wid, q, t + _NBUF], wring.at[b],
                                     tsem.at[b])
                def _drain_prev():
                    pltpu.make_async_copy(rows.at[pb],
                                          acc_sh.at[dring.at[pb]],
                                          ssem.at[pb]).wait()
                if b == 0:
                    @pl.when(to > 0)
                    def _():
                        _drain_prev()
                else:
                    _drain_prev()
                @pl.when(t + _NBUF - 1 < _BCAP)
                def _():
                    pltpu.async_copy(dt.at[wid, q, t + _NBUF - 1],
                                     dring.at[pb], dsem.at[pb])
                pltpu.make_async_copy(dt.at[wid, q, t], dring.at[b],
                                      dsem.at[b]).wait()
                pltpu.async_copy(rows.at[b], acc_sh.at[dring.at[b]],
                                 ssem.at[b], add=True)
                @pl.when(t + _NBUF - 1 < _BCAP)
                def _():
                    pltpu.make_async_copy(st.at[wid, q, pl.ds(0, 2)],
                                          sring.at[pl.ds(0, 2)],
                                          tsem.at[pb]).wait()
                    pltpu.async_copy(hstage.at[sring.at[pb]], rows.at[pb],
                                     gsem.at[pb])
            return carry
        lax.fori_loop(0, _BCAP // _NBUF, _outer, 0)

        pltpu.make_async_copy(rows.at[(_BCAP - 1) % _NBUF],
                              acc_sh.at[dring.at[(_BCAP - 1) % _NBUF]],
                              ssem.at[(_BCAP - 1) % _NBUF]).wait()
        plsc.subcore_barrier()

    pltpu.sync_copy(acc_sh.at[pl.ds(s * _RPT, _RPT)],
                    out_hbm.at[c, pl.ds(s * _RPT, _RPT)])


def _make_spmm():
    mesh = plsc.VectorSubcoreMesh(core_axis_name="c", subcore_axis_name="s")
    return pl.kernel(
        _spmm_body,
        out_type=jax.ShapeDtypeStruct((_NC, _NPAD, _NHID), jnp.float32),
        mesh=mesh,
        scratch_types=[
            pltpu.VMEM((_NBUF, _CHUNK), jnp.int32),
            pltpu.VMEM((_NBUF, _CHUNK), jnp.float32),
            pltpu.VMEM((_NBUF, _CHUNK), jnp.int32),
            pltpu.VMEM((_NBUF, _CHUNK, _NHID), jnp.float32),
            pltpu.VMEM_SHARED((_NPAD, _NHID), jnp.float32),
            pltpu.VMEM_SHARED((_BROWS, _NHID), jnp.float32),
            pltpu.SemaphoreType.DMA((_NBUF,)),
            pltpu.SemaphoreType.DMA((_NBUF,)),
            pltpu.SemaphoreType.DMA((_NBUF,)),
            pltpu.SemaphoreType.DMA((_NBUF,)),
            pltpu.SemaphoreType.DMA,
        ],
    )


_BN = 1024
_BNF = 1000


def _fc0_body(x_ref, w_ref, b_ref, o_ref):
    t = jnp.dot(x_ref[...], w_ref[...], preferred_element_type=jnp.float32)
    o_ref[...] = jnp.maximum(t + b_ref[...], 0.0)


def _dense_body(theta, p_ref, h0_ref, h_ref, w_ref, o_ref):
    sup = (1.0 - _ALPHA) * (p_ref[0] + p_ref[1]) + _ALPHA * h0_ref[...]
    t = jnp.dot(sup, w_ref[...], preferred_element_type=jnp.float32)
    o_ref[...] = jnp.maximum(theta * t + (1.0 - theta) * sup + h_ref[...], 0.0)


def _dense_final_body(theta, p_ref, h0_ref, h_ref, w_ref, wo_ref, bo_ref,
                      o_ref):
    sup = (1.0 - _ALPHA) * (p_ref[0] + p_ref[1]) + _ALPHA * h0_ref[...]
    t = jnp.dot(sup, w_ref[...], preferred_element_type=jnp.float32)
    hn = jnp.maximum(theta * t + (1.0 - theta) * sup + h_ref[...], 0.0)
    t2 = jnp.dot(hn, wo_ref[...], preferred_element_type=jnp.float32)
    o_ref[...] = jax.nn.sigmoid(t2 + bo_ref[...])


def _dense_final(p, h0, h, W, theta, Wout, bout):
    return pl.pallas_call(
        functools.partial(_dense_final_body, theta),
        grid=(_N // _BNF,),
        in_specs=[
            pl.BlockSpec((_NC, _BNF, _NHID), lambda i: (0, i, 0)),
            pl.BlockSpec((_BNF, _NHID), lambda i: (i, 0)),
            pl.BlockSpec((_BNF, _NHID), lambda i: (i, 0)),
            pl.BlockSpec((_NHID, _NHID), lambda i: (0, 0)),
            pl.BlockSpec((_NHID, _NCLASS), lambda i: (0, 0)),
            pl.BlockSpec((1, _NCLASS), lambda i: (0, 0)),
        ],
        out_specs=pl.BlockSpec((_BNF, _NCLASS), lambda i: (i, 0)),
        out_shape=jax.ShapeDtypeStruct((_N, _NCLASS), jnp.float32),
    )(p, h0, h, W, Wout, bout.reshape(1, _NCLASS))


def _fc0(x, W0, b0):
    return pl.pallas_call(
        _fc0_body,
        grid=(_NPAD // _BN,),
        in_specs=[
            pl.BlockSpec((_BN, _NFEAT), lambda i: (i, 0)),
            pl.BlockSpec((_NFEAT, _NHID), lambda i: (0, 0)),
            pl.BlockSpec((1, _NHID), lambda i: (0, 0)),
        ],
        out_specs=pl.BlockSpec((_BN, _NHID), lambda i: (i, 0)),
        out_shape=jax.ShapeDtypeStruct((_NPAD, _NHID), jnp.float32),
    )(x, W0, b0.reshape(1, _NHID))


def _dense(p, h0, h, W, theta):
    return pl.pallas_call(
        functools.partial(_dense_body, theta),
        grid=(_NPAD // _BN,),
        in_specs=[
            pl.BlockSpec((_NC, _BN, _NHID), lambda i: (0, i, 0)),
            pl.BlockSpec((_BN, _NHID), lambda i: (i, 0)),
            pl.BlockSpec((_BN, _NHID), lambda i: (i, 0)),
            pl.BlockSpec((_NHID, _NHID), lambda i: (0, 0)),
        ],
        out_specs=pl.BlockSpec((_BN, _NHID), lambda i: (i, 0)),
        out_shape=jax.ShapeDtypeStruct((_NPAD, _NHID), jnp.float32),
    )(p, h0, h, W)


def kernel(x, edge_index, edge_weight, W0, b0, Wc, Wout, bout):
    ppt = _EPT - _E // _NW
    sp = jnp.concatenate(
        [edge_index[0].reshape(_NW, _E // _NW),
         jnp.full((_NW, ppt), _NPAD - 1, jnp.int32)], axis=1)
    dp = jnp.concatenate(
        [edge_index[1].reshape(_NW, _E // _NW),
         jnp.zeros((_NW, ppt), jnp.int32)], axis=1)
    wpf = jnp.concatenate(
        [edge_weight.reshape(_NW, _E // _NW),
         jnp.zeros((_NW, ppt), jnp.float32)], axis=1)
    st_f, dt_f, wt_f = _make_presort()(sp, dp, wpf)
    st = st_f.reshape(_NW, _NBKT, _BCAP, _CHUNK)
    dt = dt_f.reshape(_NW, _NBKT, _BCAP, _CHUNK)
    wt = wt_f.reshape(_NW, _NBKT, _BCAP, _CHUNK)

    xpad = jnp.concatenate(
        [x, jnp.zeros((_NPAD - _N, _NFEAT), jnp.float32)], axis=0)
    spmm = _make_spmm()
    h = _fc0(xpad, W0, b0)
    h0 = h
    for i in range(_NLAYERS - 1):
        theta = math.log(_LAMDA / (i + 1) + 1.0)
        p = spmm(h, st, wt, dt)
        h = _dense(p, h0, h, Wc[i], theta)
    theta = math.log(_LAMDA / _NLAYERS + 1.0)
    p = spmm(h, st, wt, dt)
    return _dense_final(p, h0, h, Wc[_NLAYERS - 1], theta, Wout, bout)
